# baseline (speedup 1.0000x reference)
import functools

import jax
import jax.numpy as jnp
from jax import lax
from jax.experimental import pallas as pl
from jax.experimental.pallas import tpu as pltpu
from jax.experimental.pallas import tpu_sc as plsc

_NC = 2
_NS = 16
_NW = _NC * _NS

_BATCH = 16384
_D = 128
_ROWS_PER_W = _BATCH // _NW
_CHUNK = 64
_CHUNKS_PER_W = _ROWS_PER_W // _CHUNK


def _sc_gather_body(uidx_hbm, iidx_hbm, t_umf, t_imf, t_umlp, t_imlp,
                    womf_hbm,
                    o_umlp, o_imlp, o_s,
                    uidx_v, iidx_v, wo_v, sdot_v,
                    rows_a, rows_b, rows_c, rows_d,
                    sem_a, sem_b, sem_c, sem_d):
  wid = lax.axis_index("s") * _NC + lax.axis_index("c")
  idx_row_base = wid * _CHUNKS_PER_W
  out_base = wid * _ROWS_PER_W

  pltpu.sync_copy(uidx_hbm.at[pl.ds(idx_row_base, _CHUNKS_PER_W)], uidx_v)
  pltpu.sync_copy(iidx_hbm.at[pl.ds(idx_row_base, _CHUNKS_PER_W)], iidx_v)
  pltpu.sync_copy(womf_hbm, wo_v)
  w_regs = [wo_v[pl.ds(16 * j, 16)] for j in range(_D // 16)]

  ubufs, ibufs = (rows_a, rows_b), (rows_c, rows_d)
  usems, isems = (sem_a, sem_b), (sem_c, sem_d)

  def start_mf(c):
    pltpu.async_copy(t_umf.at[uidx_v.at[c]], ubufs[c % 2], usems[c % 2])
    pltpu.async_copy(t_imf.at[iidx_v.at[c]], ibufs[c % 2], isems[c % 2])

  start_mf(0)
  for c in range(_CHUNKS_PER_W):
    if c + 1 < _CHUNKS_PER_W:
      start_mf(c + 1)
    pltpu.make_async_copy(
        t_umf.at[uidx_v.at[c]], ubufs[c % 2], usems[c % 2]).wait()
    pltpu.make_async_copy(
        t_imf.at[iidx_v.at[c]], ibufs[c % 2], isems[c % 2]).wait()
    ub, ib = ubufs[c % 2], ibufs[c % 2]

    def row_body(r, _, ub=ub, ib=ib, c=c):
      acc = ub[r, pl.ds(0, 16)] * ib[r, pl.ds(0, 16)] * w_regs[0]
      for j in range(1, _D // 16):
        acc += ub[r, pl.ds(16 * j, 16)] * ib[r, pl.ds(16 * j, 16)] * w_regs[j]
      sdot_v[c * _CHUNK + r, :] = acc
      return 0

    lax.fori_loop(0, _CHUNK, row_body, 0)

  rounds = []
  for tab, idx_v, out in ((t_umlp, uidx_v, o_umlp), (t_imlp, iidx_v, o_imlp)):
    for j in range(_CHUNKS_PER_W):
      rounds.append((tab, idx_v, j, out))

  bufs = (rows_a, rows_b)
  sems = (sem_a, sem_b)

  def start(r):
    tab, idx_v, j, _ = rounds[r]
    pltpu.async_copy(tab.at[idx_v.at[j]], bufs[r % 2], sems[r % 2])

  start(0)
  for r in range(len(rounds)):
    if r + 1 < len(rounds):
      start(r + 1)
    _, _, j, out = rounds[r]
    pltpu.make_async_copy(
        rounds[r][0].at[rounds[r][1].at[j]], bufs[r % 2], sems[r % 2]).wait()
    pltpu.sync_copy(bufs[r % 2], out.at[pl.ds(out_base + j * _CHUNK, _CHUNK)])

  pltpu.sync_copy(sdot_v, o_s.at[pl.ds(out_base, _ROWS_PER_W)])


def _sc_gather(uidx2d, iidx2d, t_umf, t_imf, t_umlp, t_imlp, womf):
  mesh = plsc.VectorSubcoreMesh(core_axis_name="c", subcore_axis_name="s",
                                num_cores=_NC, num_subcores=_NS)
  out = jax.ShapeDtypeStruct((_BATCH, _D), jnp.float32)
  out_s = jax.ShapeDtypeStruct((_BATCH, 16), jnp.float32)
  k = pl.kernel(
      _sc_gather_body,
      out_type=(out, out, out_s),
      mesh=mesh,
      scratch_types=[
          pltpu.VMEM((_CHUNKS_PER_W, _CHUNK), jnp.int32),
          pltpu.VMEM((_CHUNKS_PER_W, _CHUNK), jnp.int32),
          pltpu.VMEM((_D,), jnp.float32),
          pltpu.VMEM((_ROWS_PER_W, 16), jnp.float32),
          pltpu.VMEM((_CHUNK, _D), jnp.float32),
          pltpu.VMEM((_CHUNK, _D), jnp.float32),
          pltpu.VMEM((_CHUNK, _D), jnp.float32),
          pltpu.VMEM((_CHUNK, _D), jnp.float32),
          pltpu.SemaphoreType.DMA,
          pltpu.SemaphoreType.DMA,
          pltpu.SemaphoreType.DMA,
          pltpu.SemaphoreType.DMA,
      ],
  )
  return k(uidx2d, iidx2d, t_umf, t_imf, t_umlp, t_imlp, womf)


_BLK = 1024


def _tc_mlp_body(umlp, imlp, s, w1a, w1b, b1, w2, b2, w3, b3,
                 womlp, bo, out):
  h = jnp.dot(umlp[...], w1a[...], preferred_element_type=jnp.float32)
  h += jnp.dot(imlp[...], w1b[...], preferred_element_type=jnp.float32)
  h = jnp.maximum(h + b1[...], 0.0)
  h = jnp.maximum(
      jnp.dot(h, w2[...], preferred_element_type=jnp.float32) + b2[...], 0.0)
  h = jnp.maximum(
      jnp.dot(h, w3[...], preferred_element_type=jnp.float32) + b3[...], 0.0)
  logit = jnp.dot(h, womlp[...], preferred_element_type=jnp.float32)[:, 0]
  logit += jnp.sum(s[...], axis=1) + bo[0, 0]
  out[...] = 1.0 / (1.0 + jnp.exp(-logit))


def _tc_mlp(umlp, imlp, s, w1a, w1b, b1, w2, b2, w3, b3, womlp, bo):
  n_blk = _BATCH // _BLK
  batch_spec = pl.BlockSpec((_BLK, _D), lambda i: (i, 0))
  s_spec = pl.BlockSpec((_BLK, 16), lambda i: (i, 0))
  vec_spec = pl.BlockSpec((_BLK,), lambda i: (i,))
  full = lambda shape: pl.BlockSpec(shape, lambda i: tuple(0 for _ in shape))
  return pl.pallas_call(
      _tc_mlp_body,
      grid=(n_blk,),
      in_specs=[
          batch_spec, batch_spec, s_spec,
          full((_D, _D)), full((_D, _D)), full((1, _D)),
          full((_D, 64)), full((1, 64)),
          full((64, 32)), full((1, 32)),
          full((32, 1)), full((1, 1)),
      ],
      out_specs=vec_spec,
      out_shape=jax.ShapeDtypeStruct((_BATCH,), jnp.float32),
  )(umlp, imlp, s, w1a, w1b, b1, w2, b2, w3, b3, womlp, bo)


@jax.jit
def kernel(user_indices, item_indices, user_emb_mf, item_emb_mf,
           user_emb_mlp, item_emb_mlp, W1, b1, W2, b2, W3, b3, Wo, bo):
  uidx2d = user_indices.reshape(_BATCH // _CHUNK, _CHUNK)
  iidx2d = item_indices.reshape(_BATCH // _CHUNK, _CHUNK)
  womf = Wo[0, :_D]
  umlp, imlp, s = _sc_gather(uidx2d, iidx2d, user_emb_mf, item_emb_mf,
                             user_emb_mlp, item_emb_mlp, womf)
  w1a = W1[:, :_D].T
  w1b = W1[:, _D:].T
  w2 = W2.T
  w3 = W3.T
  womlp = Wo[0, _D:].reshape(32, 1)
  return _tc_mlp(umlp, imlp, s, w1a, w1b, b1.reshape(1, _D),
                 w2, b2.reshape(1, 64), w3, b3.reshape(1, 32),
                 womlp, bo.reshape(1, 1))
